# Initial kernel scaffold; baseline (speedup 1.0000x reference)
import jax
import jax.numpy as jnp
from jax.experimental import pallas as pl


def kernel(out_gnn, batch_input, W1, b1, W2, b2):
    raise NotImplementedError("write your pallas kernel here")


# speedup vs baseline: 5.3966x; 5.3966x over previous
import jax
import jax.numpy as jnp
from jax import lax
from jax.experimental import pallas as pl
from jax.experimental.pallas import tpu as pltpu

G = 512
D = 128
N = 100000
NB = 50
Nb = N // NB


def _score_body(e1_ref, e2_ref, w1_ref, b1_ref, w2_ref, b2_ref, raw_ref, gmax_ref):
    i = pl.program_id(0)
    e1 = e1_ref[...]
    e2 = e2_ref[...]
    wa = w1_ref[0:D, :] + w1_ref[D:2 * D, :]
    wb = w1_ref[D:2 * D, :]
    h = jnp.dot(e1, wa, preferred_element_type=jnp.float32)
    h = h - jnp.dot(e2, wb, preferred_element_type=jnp.float32)
    h = jnp.maximum(h + b1_ref[...], 0.0)
    raw = jnp.dot(h, w2_ref[...], preferred_element_type=jnp.float32) + b2_ref[...]
    raw_ref[...] = raw
    bmax = jnp.max(raw).reshape(1, 1)

    @pl.when(i == 0)
    def _init():
        gmax_ref[...] = bmax

    @pl.when(i != 0)
    def _acc():
        gmax_ref[...] = jnp.maximum(gmax_ref[...], bmax)


def _expsum_body(raw_ref, gmax_ref, seg_ref, s_ref):
    i = pl.program_id(0)
    e = jnp.exp(raw_ref[...] - gmax_ref[...])
    ids = lax.broadcasted_iota(jnp.int32, (Nb, G), 1)
    oh = (seg_ref[...] == ids).astype(jnp.float32)
    s_blk = lax.dot_general(oh, e, (((0,), (0,)), ((), ())),
                            preferred_element_type=jnp.float32)

    @pl.when(i == 0)
    def _init():
        s_ref[...] = s_blk

    @pl.when(i != 0)
    def _acc():
        s_ref[...] = s_ref[...] + s_blk


def _pool_body(e1_ref, e2_ref, raw_ref, gmax_ref, seg_ref, s_ref, out_ref):
    i = pl.program_id(0)
    e = jnp.exp(raw_ref[...] - gmax_ref[...])
    ids = lax.broadcasted_iota(jnp.int32, (Nb, G), 1)
    oh = (seg_ref[...] == ids).astype(jnp.float32)
    sg = jnp.dot(oh, s_ref[...], preferred_element_type=jnp.float32)
    att = jnp.where(sg > 0.0, e / sg, 0.0)
    wd = (e1_ref[...] - e2_ref[...]) * att
    o_blk = lax.dot_general(oh, wd, (((0,), (0,)), ((), ())),
                            preferred_element_type=jnp.float32)

    @pl.when(i == 0)
    def _init():
        out_ref[...] = o_blk

    @pl.when(i != 0)
    def _acc():
        out_ref[...] = out_ref[...] + o_blk


def kernel(out_gnn, batch_input, W1, b1, W2, b2):
    seg = batch_input[:N].reshape(N, 1)
    b1r = b1.reshape(1, D)
    b2r = b2.reshape(1, 1)

    raw, gmax = pl.pallas_call(
        _score_body,
        grid=(NB,),
        in_specs=[
            pl.BlockSpec((Nb, D), lambda i: (i, 0)),
            pl.BlockSpec((Nb, D), lambda i: (i + NB, 0)),
            pl.BlockSpec((2 * D, D), lambda i: (0, 0)),
            pl.BlockSpec((1, D), lambda i: (0, 0)),
            pl.BlockSpec((D, 1), lambda i: (0, 0)),
            pl.BlockSpec((1, 1), lambda i: (0, 0)),
        ],
        out_specs=[
            pl.BlockSpec((Nb, 1), lambda i: (i, 0)),
            pl.BlockSpec((1, 1), lambda i: (0, 0)),
        ],
        out_shape=[
            jax.ShapeDtypeStruct((N, 1), jnp.float32),
            jax.ShapeDtypeStruct((1, 1), jnp.float32),
        ],
    )(out_gnn, out_gnn, W1, b1r, W2, b2r)

    s = pl.pallas_call(
        _expsum_body,
        grid=(NB,),
        in_specs=[
            pl.BlockSpec((Nb, 1), lambda i: (i, 0)),
            pl.BlockSpec((1, 1), lambda i: (0, 0)),
            pl.BlockSpec((Nb, 1), lambda i: (i, 0)),
        ],
        out_specs=pl.BlockSpec((G, 1), lambda i: (0, 0)),
        out_shape=jax.ShapeDtypeStruct((G, 1), jnp.float32),
    )(raw, gmax, seg)

    out = pl.pallas_call(
        _pool_body,
        grid=(NB,),
        in_specs=[
            pl.BlockSpec((Nb, D), lambda i: (i, 0)),
            pl.BlockSpec((Nb, D), lambda i: (i + NB, 0)),
            pl.BlockSpec((Nb, 1), lambda i: (i, 0)),
            pl.BlockSpec((1, 1), lambda i: (0, 0)),
            pl.BlockSpec((Nb, 1), lambda i: (i, 0)),
            pl.BlockSpec((G, 1), lambda i: (0, 0)),
        ],
        out_specs=pl.BlockSpec((G, D), lambda i: (0, 0)),
        out_shape=jax.ShapeDtypeStruct((G, D), jnp.float32),
    )(out_gnn, out_gnn, raw, gmax, seg, s)

    return out


# speedup vs baseline: 9.8632x; 1.8277x over previous
import jax
import jax.numpy as jnp
from jax import lax
from jax.experimental import pallas as pl
from jax.experimental.pallas import tpu as pltpu

G = 512
D = 128
N = 100000
NB = 50
Nb = N // NB


def _fused_body(e1_ref, e2_ref, w1_ref, b1_ref, w2_ref, b2_ref, seg_ref,
                out_ref, num_ref, den_ref, m_ref):
    i = pl.program_id(0)
    e1 = e1_ref[...]
    e2 = e2_ref[...]
    wa = w1_ref[0:D, :] + w1_ref[D:2 * D, :]
    wb = w1_ref[D:2 * D, :]
    h = jnp.dot(e1, wa, preferred_element_type=jnp.float32)
    h = h - jnp.dot(e2, wb, preferred_element_type=jnp.float32)
    h = jnp.maximum(h + b1_ref[...], 0.0)
    raw = jnp.dot(h, w2_ref[...], preferred_element_type=jnp.float32) + b2_ref[...]

    m_b = jnp.max(raw)
    e = jnp.exp(raw - m_b)
    ids = lax.broadcasted_iota(jnp.int32, (Nb, G), 1)
    oh = (seg_ref[...] == ids).astype(jnp.bfloat16)
    wd = ((e1 - e2) * e).astype(jnp.bfloat16)
    nb = lax.dot_general(oh, wd, (((0,), (0,)), ((), ())),
                         preferred_element_type=jnp.float32)
    db = lax.dot_general(oh, e.astype(jnp.bfloat16), (((0,), (0,)), ((), ())),
                         preferred_element_type=jnp.float32)

    @pl.when(i == 0)
    def _init():
        m_ref[0] = m_b
        num_ref[...] = nb
        den_ref[...] = db

    @pl.when(i != 0)
    def _acc():
        m_old = m_ref[0]
        m_new = jnp.maximum(m_old, m_b)
        alpha = jnp.exp(m_old - m_new)
        beta = jnp.exp(m_b - m_new)
        num_ref[...] = num_ref[...] * alpha + nb * beta
        den_ref[...] = den_ref[...] * alpha + db * beta
        m_ref[0] = m_new

    @pl.when(i == NB - 1)
    def _final():
        den = den_ref[...]
        out_ref[...] = num_ref[...] * jnp.where(den > 0.0, 1.0 / den, 0.0)


def kernel(out_gnn, batch_input, W1, b1, W2, b2):
    seg = batch_input[:N].reshape(N, 1)
    b1r = b1.reshape(1, D)
    b2r = b2.reshape(1, 1)

    out = pl.pallas_call(
        _fused_body,
        grid=(NB,),
        in_specs=[
            pl.BlockSpec((Nb, D), lambda i: (i, 0)),
            pl.BlockSpec((Nb, D), lambda i: (i + NB, 0)),
            pl.BlockSpec((2 * D, D), lambda i: (0, 0)),
            pl.BlockSpec((1, D), lambda i: (0, 0)),
            pl.BlockSpec((D, 1), lambda i: (0, 0)),
            pl.BlockSpec((1, 1), lambda i: (0, 0)),
            pl.BlockSpec((Nb, 1), lambda i: (i, 0)),
        ],
        out_specs=pl.BlockSpec((G, D), lambda i: (0, 0)),
        out_shape=jax.ShapeDtypeStruct((G, D), jnp.float32),
        scratch_shapes=[
            pltpu.VMEM((G, D), jnp.float32),
            pltpu.VMEM((G, 1), jnp.float32),
            pltpu.SMEM((1,), jnp.float32),
        ],
    )(out_gnn, out_gnn, W1, b1r, W2, b2r, seg)

    return out


# speedup vs baseline: 10.8864x; 1.1037x over previous
import jax
import jax.numpy as jnp
from jax import lax
from jax.experimental import pallas as pl
from jax.experimental.pallas import tpu as pltpu

G = 512
D = 128
N = 100000
NB = 25
Nb = N // NB


def _fused_body(e1_ref, e2_ref, w1_ref, b1_ref, w2_ref, b2_ref, seg_ref,
                out_ref, num_ref, den_ref, m_ref):
    i = pl.program_id(0)
    e1 = e1_ref[...]
    e2 = e2_ref[...]
    wa = w1_ref[0:D, :] + w1_ref[D:2 * D, :]
    wb = w1_ref[D:2 * D, :]
    h = jnp.dot(e1, wa, preferred_element_type=jnp.float32)
    h = h - jnp.dot(e2, wb, preferred_element_type=jnp.float32)
    h = jnp.maximum(h + b1_ref[...], 0.0)
    raw = jnp.dot(h, w2_ref[...], preferred_element_type=jnp.float32) + b2_ref[...]

    m_b = jnp.max(raw)
    e = jnp.exp(raw - m_b)
    ids = lax.broadcasted_iota(jnp.int32, (Nb, G), 1)
    oh = (seg_ref[...] == ids).astype(jnp.bfloat16)
    wd = ((e1 - e2) * e).astype(jnp.bfloat16)
    nb = lax.dot_general(oh, wd, (((0,), (0,)), ((), ())),
                         preferred_element_type=jnp.float32)
    db = lax.dot_general(oh, e.astype(jnp.bfloat16), (((0,), (0,)), ((), ())),
                         preferred_element_type=jnp.float32)

    @pl.when(i == 0)
    def _init():
        m_ref[0] = m_b
        num_ref[...] = nb
        den_ref[...] = db

    @pl.when(i != 0)
    def _acc():
        m_old = m_ref[0]
        m_new = jnp.maximum(m_old, m_b)
        alpha = jnp.exp(m_old - m_new)
        beta = jnp.exp(m_b - m_new)
        num_ref[...] = num_ref[...] * alpha + nb * beta
        den_ref[...] = den_ref[...] * alpha + db * beta
        m_ref[0] = m_new

    @pl.when(i == NB - 1)
    def _final():
        den = den_ref[...]
        out_ref[...] = num_ref[...] * jnp.where(den > 0.0, 1.0 / den, 0.0)


def kernel(out_gnn, batch_input, W1, b1, W2, b2):
    seg = batch_input[:N].reshape(N, 1)
    b1r = b1.reshape(1, D)
    b2r = b2.reshape(1, 1)

    out = pl.pallas_call(
        _fused_body,
        grid=(NB,),
        in_specs=[
            pl.BlockSpec((Nb, D), lambda i: (i, 0)),
            pl.BlockSpec((Nb, D), lambda i: (i + NB, 0)),
            pl.BlockSpec((2 * D, D), lambda i: (0, 0)),
            pl.BlockSpec((1, D), lambda i: (0, 0)),
            pl.BlockSpec((D, 1), lambda i: (0, 0)),
            pl.BlockSpec((1, 1), lambda i: (0, 0)),
            pl.BlockSpec((Nb, 1), lambda i: (i, 0)),
        ],
        out_specs=pl.BlockSpec((G, D), lambda i: (0, 0)),
        out_shape=jax.ShapeDtypeStruct((G, D), jnp.float32),
        scratch_shapes=[
            pltpu.VMEM((G, D), jnp.float32),
            pltpu.VMEM((G, 1), jnp.float32),
            pltpu.SMEM((1,), jnp.float32),
        ],
    )(out_gnn, out_gnn, W1, b1r, W2, b2r, seg)

    return out


# speedup vs baseline: 15.9365x; 1.4639x over previous
import jax
import jax.numpy as jnp
from jax import lax
from jax.experimental import pallas as pl
from jax.experimental.pallas import tpu as pltpu

G = 512
D = 128
N = 100000
NB = 25
Nb = N // NB


def _fused_body(e1_ref, e2_ref, w1_ref, b1_ref, w2_ref, b2_ref, seg_ref,
                out_ref, num_ref, den_ref, m_ref):
    i = pl.program_id(0)
    e1 = e1_ref[...]
    e2 = e2_ref[...]
    wa = w1_ref[0:D, :] + w1_ref[D:2 * D, :]
    wb = w1_ref[D:2 * D, :]
    h = jnp.dot(e1, wa, preferred_element_type=jnp.float32)
    h = h - jnp.dot(e2, wb, preferred_element_type=jnp.float32)
    h = jnp.maximum(h + b1_ref[...], 0.0)
    raw = jnp.dot(h, w2_ref[...], preferred_element_type=jnp.float32) + b2_ref[...]

    m_b = jnp.max(raw)
    e = jnp.exp(raw - m_b)
    seg_row = seg_ref[0]
    ids = lax.broadcasted_iota(jnp.int32, (G, Nb), 0)
    ohT = (seg_row == ids).astype(jnp.bfloat16)
    wd = ((e1 - e2) * e).astype(jnp.bfloat16)
    nb = jnp.dot(ohT, wd, preferred_element_type=jnp.float32)
    db = jnp.dot(ohT, e.astype(jnp.bfloat16),
                 preferred_element_type=jnp.float32)

    @pl.when(i == 0)
    def _init():
        m_ref[0] = m_b
        num_ref[...] = nb
        den_ref[...] = db

    @pl.when(i != 0)
    def _acc():
        m_old = m_ref[0]
        m_new = jnp.maximum(m_old, m_b)
        alpha = jnp.exp(m_old - m_new)
        beta = jnp.exp(m_b - m_new)
        num_ref[...] = num_ref[...] * alpha + nb * beta
        den_ref[...] = den_ref[...] * alpha + db * beta
        m_ref[0] = m_new

    @pl.when(i == NB - 1)
    def _final():
        den = den_ref[...]
        out_ref[...] = num_ref[...] * jnp.where(den > 0.0, 1.0 / den, 0.0)


def kernel(out_gnn, batch_input, W1, b1, W2, b2):
    seg = batch_input[:N].reshape(NB, 1, Nb)
    b1r = b1.reshape(1, D)
    b2r = b2.reshape(1, 1)

    out = pl.pallas_call(
        _fused_body,
        grid=(NB,),
        in_specs=[
            pl.BlockSpec((Nb, D), lambda i: (i, 0)),
            pl.BlockSpec((Nb, D), lambda i: (i + NB, 0)),
            pl.BlockSpec((2 * D, D), lambda i: (0, 0)),
            pl.BlockSpec((1, D), lambda i: (0, 0)),
            pl.BlockSpec((D, 1), lambda i: (0, 0)),
            pl.BlockSpec((1, 1), lambda i: (0, 0)),
            pl.BlockSpec((1, 1, Nb), lambda i: (i, 0, 0)),
        ],
        out_specs=pl.BlockSpec((G, D), lambda i: (0, 0)),
        out_shape=jax.ShapeDtypeStruct((G, D), jnp.float32),
        scratch_shapes=[
            pltpu.VMEM((G, D), jnp.float32),
            pltpu.VMEM((G, 1), jnp.float32),
            pltpu.SMEM((1,), jnp.float32),
        ],
    )(out_gnn, out_gnn, W1, b1r, W2, b2r, seg)

    return out
